# Initial kernel scaffold; baseline (speedup 1.0000x reference)
import jax
import jax.numpy as jnp
from jax.experimental import pallas as pl


def kernel(inputs, table):
    raise NotImplementedError("write your pallas kernel here")


# speedup vs baseline: 1.4422x; 1.4422x over previous
import functools

import jax
import jax.numpy as jnp
from jax import lax
from jax.experimental import pallas as pl
from jax.experimental.pallas import tpu as pltpu
from jax.experimental.pallas import tpu_sc as plsc

NUM_ROWS = 16384 * 26
DIM = 32
NC, NS = 2, 16
NW = NC * NS
ROWS_PER_W = NUM_ROWS // NW
CHUNK = 128
NCHUNK = ROWS_PER_W // CHUNK


def _body(idx_hbm, table_hbm, out_hbm, idx_v, rows_v, sem):
    w = lax.axis_index("s") * NC + lax.axis_index("c")
    pltpu.sync_copy(idx_hbm.at[w], idx_v)
    base = w * ROWS_PER_W

    def step(j):
        pltpu.async_copy(table_hbm.at[idx_v.at[j]], rows_v, sem).wait()
        pltpu.sync_copy(rows_v, out_hbm.at[pl.ds(base + j * CHUNK, CHUNK)])

    pl.loop(0, NCHUNK)(step)


@functools.partial(jax.jit, static_argnums=())
def _run(idx3, table):
    k = pl.kernel(
        _body,
        out_type=jax.ShapeDtypeStruct((NUM_ROWS, DIM), jnp.float32),
        mesh=plsc.VectorSubcoreMesh(core_axis_name="c", subcore_axis_name="s"),
        scratch_types=[
            pltpu.VMEM((NCHUNK, CHUNK), jnp.int32),
            pltpu.VMEM((CHUNK, DIM), jnp.float32),
            pltpu.SemaphoreType.DMA,
        ],
        compiler_params=pltpu.CompilerParams(use_tc_tiling_on_sc=False),
    )
    return k(idx3, table)


def kernel(inputs, table):
    idx3 = inputs.astype(jnp.int32).reshape(NW, NCHUNK, CHUNK)
    out = _run(idx3, table)
    return out.reshape(inputs.shape + (DIM,))


# speedup vs baseline: 1.5755x; 1.0924x over previous
import functools

import jax
import jax.numpy as jnp
from jax import lax
from jax.experimental import pallas as pl
from jax.experimental.pallas import tpu as pltpu
from jax.experimental.pallas import tpu_sc as plsc

NUM_ROWS = 16384 * 26
DIM = 32
NC, NS = 2, 16
NW = NC * NS
ROWS_PER_W = NUM_ROWS // NW
CHUNK = 128
NCHUNK = ROWS_PER_W // CHUNK
GROUP = 4
GROWS = GROUP * CHUNK
NGROUP = NCHUNK // GROUP
NBUF = 2
MAIN = NGROUP - NBUF


def _body(idx_hbm, table_hbm, out_hbm, idx_v, rows_v, g0, g1, s0, s1):
    gsem = (g0, g1)
    ssem = (s0, s1)
    w = lax.axis_index("s") * NC + lax.axis_index("c")
    pltpu.sync_copy(idx_hbm.at[w], idx_v)
    base = w * ROWS_PER_W

    def fire(gi, p):
        for b in range(GROUP):
            pltpu.async_copy(
                table_hbm.at[idx_v.at[gi * GROUP + b]],
                rows_v.at[p].at[pl.ds(b * CHUNK, CHUNK)],
                gsem[p])

    def drain(gi, p):
        for b in range(GROUP):
            pltpu.make_async_copy(
                table_hbm.at[idx_v.at[gi * GROUP + b]],
                rows_v.at[p].at[pl.ds(b * CHUNK, CHUNK)],
                gsem[p]).wait()

    def process(gi, p, fire_next):
        drain(gi, p)
        st = pltpu.async_copy(
            rows_v.at[p], out_hbm.at[pl.ds(base + gi * GROWS, GROWS)], ssem[p])
        st.wait()
        if fire_next:
            fire(gi + NBUF, p)

    for p in range(NBUF):
        fire(p, p)

    def grp(g):
        for p in range(NBUF):
            process(g + p, p, True)

    pl.loop(0, MAIN, step=NBUF)(grp)

    for p in range(NBUF):
        process(MAIN + p, p, False)


@functools.partial(jax.jit, static_argnums=())
def _run(idx3, table):
    k = pl.kernel(
        _body,
        out_type=jax.ShapeDtypeStruct((NUM_ROWS, DIM), jnp.float32),
        mesh=plsc.VectorSubcoreMesh(core_axis_name="c", subcore_axis_name="s"),
        scratch_types=[
            pltpu.VMEM((NCHUNK, CHUNK), jnp.int32),
            pltpu.VMEM((NBUF, GROWS, DIM), jnp.float32),
            pltpu.SemaphoreType.DMA,
            pltpu.SemaphoreType.DMA,
            pltpu.SemaphoreType.DMA,
            pltpu.SemaphoreType.DMA,
        ],
        compiler_params=pltpu.CompilerParams(use_tc_tiling_on_sc=False),
    )
    return k(idx3, table)


def kernel(inputs, table):
    idx3 = inputs.astype(jnp.int32).reshape(NW, NCHUNK, CHUNK)
    out = _run(idx3, table)
    return out.reshape(inputs.shape + (DIM,))


# speedup vs baseline: 1.5758x; 1.0002x over previous
import functools

import jax
import jax.numpy as jnp
from jax import lax
from jax.experimental import pallas as pl
from jax.experimental.pallas import tpu as pltpu
from jax.experimental.pallas import tpu_sc as plsc

NUM_ROWS = 16384 * 26
DIM = 32
NC, NS = 2, 16
NW = NC * NS
ROWS_PER_W = NUM_ROWS // NW
CHUNK = 512
NCHUNK = ROWS_PER_W // CHUNK
GROUP = 1
GROWS = GROUP * CHUNK
NGROUP = NCHUNK // GROUP
NBUF = 2
MAIN = NGROUP - NBUF


def _body(idx_hbm, table_hbm, out_hbm, idx_v, rows_v, g0, g1, s0, s1):
    gsem = (g0, g1)
    ssem = (s0, s1)
    w = lax.axis_index("s") * NC + lax.axis_index("c")
    pltpu.sync_copy(idx_hbm.at[w], idx_v)
    base = w * ROWS_PER_W

    def fire(gi, p):
        for b in range(GROUP):
            pltpu.async_copy(
                table_hbm.at[idx_v.at[gi * GROUP + b]],
                rows_v.at[p].at[pl.ds(b * CHUNK, CHUNK)],
                gsem[p])

    def drain(gi, p):
        for b in range(GROUP):
            pltpu.make_async_copy(
                table_hbm.at[idx_v.at[gi * GROUP + b]],
                rows_v.at[p].at[pl.ds(b * CHUNK, CHUNK)],
                gsem[p]).wait()

    def process(gi, p, fire_next):
        drain(gi, p)
        st = pltpu.async_copy(
            rows_v.at[p], out_hbm.at[pl.ds(base + gi * GROWS, GROWS)], ssem[p])
        st.wait()
        if fire_next:
            fire(gi + NBUF, p)

    for p in range(NBUF):
        fire(p, p)

    def grp(g):
        for p in range(NBUF):
            process(g + p, p, True)

    pl.loop(0, MAIN, step=NBUF)(grp)

    for p in range(NBUF):
        process(MAIN + p, p, False)


@functools.partial(jax.jit, static_argnums=())
def _run(idx3, table):
    k = pl.kernel(
        _body,
        out_type=jax.ShapeDtypeStruct((NUM_ROWS, DIM), jnp.float32),
        mesh=plsc.VectorSubcoreMesh(core_axis_name="c", subcore_axis_name="s"),
        scratch_types=[
            pltpu.VMEM((NCHUNK, CHUNK), jnp.int32),
            pltpu.VMEM((NBUF, GROWS, DIM), jnp.float32),
            pltpu.SemaphoreType.DMA,
            pltpu.SemaphoreType.DMA,
            pltpu.SemaphoreType.DMA,
            pltpu.SemaphoreType.DMA,
        ],
        compiler_params=pltpu.CompilerParams(use_tc_tiling_on_sc=False),
    )
    return k(idx3, table)


def kernel(inputs, table):
    idx3 = inputs.astype(jnp.int32).reshape(NW, NCHUNK, CHUNK)
    out = _run(idx3, table)
    return out.reshape(inputs.shape + (DIM,))
